# baseline (speedup 1.0000x reference)
import functools

import jax
import jax.numpy as jnp
from jax import lax
from jax.experimental import pallas as pl
from jax.experimental.pallas import tpu as pltpu
from jax.experimental.pallas import tpu_sc as plsc

N_DEPTH = 7
EMBED_DIM = 16
N_POINTS = 16384
LANES = 16

NUM_CORES = 2
NUM_SUBCORES = 16
NUM_WORKERS = NUM_CORES * NUM_SUBCORES
PTS_PER_WORKER = N_POINTS // NUM_WORKERS
CHUNK = 128
NUM_CHUNKS = PTS_PER_WORKER // CHUNK
GROUPS_PER_CHUNK = CHUNK // LANES

_ONE_MINUS_EPS = 1.0 - 1e-06


def _body(inp_hbm, w1, w2, w3, w4, w5, w6, w7, out_hbm,
          coords_v, idx_v, rows_v, gsem):
    tables = [w1, w2, w3, w4, w5, w6, w7]
    wid = lax.axis_index("s") * NUM_CORES + lax.axis_index("c")
    base = wid * PTS_PER_WORKER

    pltpu.sync_copy(inp_hbm.at[pl.ds(base, PTS_PER_WORKER), :], coords_v)

    iota = lax.iota(jnp.int32, LANES)
    col = [jnp.full((LANES,), a, jnp.int32) for a in range(3)]

    for c in range(NUM_CHUNKS):
        for i in range(GROUPS_PER_CHUNK):
            row0 = (c * GROUPS_PER_CHUNK + i) * LANES
            ridx = iota + row0
            cell = []
            for a in range(3):
                v = plsc.load_gather(coords_v, [ridx, col[a]])
                s = (v + 1.0) * 0.5
                s = jnp.minimum(jnp.maximum(s, 0.0), _ONE_MINUS_EPS)
                cell.append((s * 128.0).astype(jnp.int32))
            x7, y7, z7 = cell
            for d in range(1, N_DEPTH + 1):
                sh = N_DEPTH - d
                flat = (((z7 >> sh) << (2 * d))
                        + ((y7 >> sh) << d)
                        + (x7 >> sh))
                idx_v[d - 1, pl.ds(i * LANES, LANES)] = flat

        copies = []
        for d in range(N_DEPTH):
            copies.append(
                pltpu.async_copy(tables[d].at[idx_v.at[d]],
                                 rows_v.at[d], gsem))
        for cp in copies:
            cp.wait()

        row_out = base + c * CHUNK
        for d in range(N_DEPTH):
            pltpu.sync_copy(
                rows_v.at[d],
                out_hbm.at[pl.ds(row_out, CHUNK),
                           pl.ds(d * EMBED_DIM, EMBED_DIM)])


@jax.jit
def kernel(input, W1, W2, W3, W4, W5, W6, W7):
    mesh = plsc.VectorSubcoreMesh(
        core_axis_name="c", subcore_axis_name="s",
        num_cores=NUM_CORES, num_subcores=NUM_SUBCORES)
    f = pl.kernel(
        _body,
        out_type=jax.ShapeDtypeStruct((N_POINTS, N_DEPTH * EMBED_DIM),
                                      jnp.float32),
        mesh=mesh,
        scratch_types=[
            pltpu.VMEM((PTS_PER_WORKER, 3), jnp.float32),
            pltpu.VMEM((N_DEPTH, CHUNK), jnp.int32),
            pltpu.VMEM((N_DEPTH, CHUNK, EMBED_DIM), jnp.float32),
            pltpu.SemaphoreType.DMA,
        ],
        compiler_params=pltpu.CompilerParams(use_tc_tiling_on_sc=False,
                                             needs_layout_passes=False),
    )
    return f(input, W1, W2, W3, W4, W5, W6, W7)


# speedup vs baseline: 5.6104x; 5.6104x over previous
import jax
import jax.numpy as jnp
from jax import lax
from jax.experimental import pallas as pl
from jax.experimental.pallas import tpu as pltpu
from jax.experimental.pallas import tpu_sc as plsc

N_DEPTH = 7
EMBED_DIM = 16
N_POINTS = 16384
OUT_DIM = N_DEPTH * EMBED_DIM
LANES = 16

NUM_CORES = 2
NUM_SUBCORES = 16
NUM_WORKERS = NUM_CORES * NUM_SUBCORES
PTS_PER_WORKER = N_POINTS // NUM_WORKERS
GROUPS = PTS_PER_WORKER // LANES

STREAM_DEPTHS = list(range(3, N_DEPTH + 1))
LOCAL_DEPTHS = [1, 2]
N_STREAM = len(STREAM_DEPTHS)

_ONE_MINUS_EPS = 1.0 - 1e-06


def _flat_tile_view(wt):
    v = wt.shape[1]
    return (wt.reshape(2, 8, v // 128, 128)
              .transpose(0, 2, 1, 3)
              .reshape(-1))


def _body(inp_t, wt1, wt2, f3, f4, f5, f6, f7, out_rm,
          coords_v, t1_v, t2_v, p3, p4, p5, p6, p7, out_v, gsem):
    flats = [f3, f4, f5, f6, f7]
    pidx = [p3, p4, p5, p6, p7]
    vocab = {d: (2 ** d) ** 3 for d in range(1, N_DEPTH + 1)}

    wid = lax.axis_index("s") * NUM_CORES + lax.axis_index("c")
    base = wid * PTS_PER_WORKER

    pltpu.sync_copy(inp_t.at[:, pl.ds(base, PTS_PER_WORKER)], coords_v)
    pltpu.sync_copy(wt1, t1_v)
    pltpu.sync_copy(wt2, t2_v)
    local_tv = {1: t1_v, 2: t2_v}

    @pl.loop(0, GROUPS)
    def _grp(i):
        p0 = i * LANES
        cell = []
        for a in range(3):
            v = coords_v[a, pl.ds(p0, LANES)]
            s = (v + 1.0) * 0.5
            s = jnp.minimum(jnp.maximum(s, 0.0), _ONE_MINUS_EPS)
            cell.append((s * 128.0).astype(jnp.int32))
        x7, y7, z7 = cell
        for d in range(1, N_DEPTH + 1):
            sh = N_DEPTH - d
            idx = (((z7 >> sh) << (2 * d))
                   + ((y7 >> sh) << d)
                   + (x7 >> sh))
            if d in LOCAL_DEPTHS:
                tv = local_tv[d]
                for e in range(EMBED_DIM):
                    row = plsc.load_gather(
                        tv, [jnp.full((LANES,), e, jnp.int32), idx])
                    out_v[pl.ds(((d - 1) * EMBED_DIM + e) * PTS_PER_WORKER
                                + p0, LANES)] = row
            else:
                vv = vocab[d]
                rpart = ((idx >> 7) << 10) + (idx & 127)
                di = STREAM_DEPTHS.index(d)
                for e in range(EMBED_DIM):
                    a_c = (e // 8) * (vv * 8) + (e % 8) * 128
                    pidx[di][pl.ds(e * PTS_PER_WORKER + p0,
                                   LANES)] = rpart + a_c

    copies = []
    for di, d in enumerate(STREAM_DEPTHS):
        for e in range(EMBED_DIM):
            row = (d - 1) * EMBED_DIM + e
            idx_ref = pidx[di].at[pl.ds(e * PTS_PER_WORKER, PTS_PER_WORKER)]
            dst = out_v.at[pl.ds(row * PTS_PER_WORKER, PTS_PER_WORKER)]
            copies.append(
                pltpu.async_copy(flats[di].at[idx_ref], dst, gsem))
    for cp in copies:
        cp.wait()

    for row in range(OUT_DIM):
        pltpu.sync_copy(
            out_v.at[pl.ds(row * PTS_PER_WORKER, PTS_PER_WORKER)],
            out_rm.at[pl.ds(row * N_POINTS + base, PTS_PER_WORKER)])


@jax.jit
def kernel(input, W1, W2, W3, W4, W5, W6, W7):
    tables = [W1, W2, W3, W4, W5, W6, W7]
    wts = [w.T for w in tables]
    flats = [_flat_tile_view(wts[d - 1]) for d in STREAM_DEPTHS]

    mesh = plsc.VectorSubcoreMesh(
        core_axis_name="c", subcore_axis_name="s",
        num_cores=NUM_CORES, num_subcores=NUM_SUBCORES)
    f = pl.kernel(
        _body,
        out_type=jax.ShapeDtypeStruct((OUT_DIM * N_POINTS,), jnp.float32),
        mesh=mesh,
        scratch_types=[
            pltpu.VMEM((3, PTS_PER_WORKER), jnp.float32),
            pltpu.VMEM((EMBED_DIM, 8), jnp.float32),
            pltpu.VMEM((EMBED_DIM, 64), jnp.float32),
        ] + [pltpu.VMEM((EMBED_DIM * PTS_PER_WORKER,), jnp.int32)
             for _ in range(N_STREAM)] + [
            pltpu.VMEM((OUT_DIM * PTS_PER_WORKER,), jnp.float32),
            pltpu.SemaphoreType.DMA,
        ],
        compiler_params=pltpu.CompilerParams(needs_layout_passes=False),
    )
    out_rm = f(input.T, wts[0], wts[1], *flats)
    return out_rm.reshape(OUT_DIM, N_POINTS).T


# speedup vs baseline: 5.7983x; 1.0335x over previous
import jax
import jax.numpy as jnp
from jax import lax
from jax.experimental import pallas as pl
from jax.experimental.pallas import tpu as pltpu
from jax.experimental.pallas import tpu_sc as plsc

N_DEPTH = 7
EMBED_DIM = 16
N_POINTS = 16384
OUT_DIM = N_DEPTH * EMBED_DIM
LANES = 16

NUM_CORES = 2
NUM_SUBCORES = 16
NUM_WORKERS = NUM_CORES * NUM_SUBCORES
PTS_PER_WORKER = N_POINTS // NUM_WORKERS
GROUPS = PTS_PER_WORKER // LANES
CT_PER_WORKER = PTS_PER_WORKER // 128
ROW_GROUPS = OUT_DIM // 8
N_COL_TILES = N_POINTS // 128

STREAM_DEPTHS = list(range(3, N_DEPTH + 1))
LOCAL_DEPTHS = [1, 2]
N_STREAM = len(STREAM_DEPTHS)

_ONE_MINUS_EPS = 1.0 - 1e-06


def _flat_tile_view(wt):
    v = wt.shape[1]
    return (wt.reshape(2, 8, v // 128, 128)
              .transpose(0, 2, 1, 3)
              .reshape(-1))


def _pos(row, ct):
    return ((row // 8) * CT_PER_WORKER + ct) * 1024 + (row % 8) * 128


def _body(inp_t, w1, wt2, f3, f4, f5, f6, f7, out_flat,
          coords_v, t1_v, t2_v, p3, p4, p5, p6, p7, out_v, gsem):
    flats = [f3, f4, f5, f6, f7]
    pidx = [p3, p4, p5, p6, p7]
    vocab = {d: (2 ** d) ** 3 for d in range(1, N_DEPTH + 1)}

    wid = lax.axis_index("s") * NUM_CORES + lax.axis_index("c")
    base = wid * PTS_PER_WORKER

    pltpu.sync_copy(inp_t.at[:, pl.ds(base, PTS_PER_WORKER)], coords_v)
    pltpu.sync_copy(w1, t1_v)
    pltpu.sync_copy(wt2, t2_v)

    @pl.loop(0, GROUPS)
    def _grp(i):
        p0 = i * LANES
        ct = i // 8
        pm = (i % 8) * LANES
        cell = []
        for a in range(3):
            v = coords_v[a, pl.ds(p0, LANES)]
            s = (v + 1.0) * 0.5
            s = jnp.minimum(jnp.maximum(s, 0.0), _ONE_MINUS_EPS)
            cell.append((s * 128.0).astype(jnp.int32))
        x7, y7, z7 = cell
        for d in range(1, N_DEPTH + 1):
            sh = N_DEPTH - d
            idx = (((z7 >> sh) << (2 * d))
                   + ((y7 >> sh) << d)
                   + (x7 >> sh))
            if d in LOCAL_DEPTHS:
                for e in range(EMBED_DIM):
                    if d == 1:
                        row_v = plsc.load_gather(
                            t1_v, [idx, jnp.full((LANES,), e, jnp.int32)])
                    else:
                        row_v = plsc.load_gather(
                            t2_v, [jnp.full((LANES,), e, jnp.int32), idx])
                    row = (d - 1) * EMBED_DIM + e
                    off = (((row // 8) * CT_PER_WORKER) * 1024
                           + (row % 8) * 128)
                    out_v[pl.ds(off + ct * 1024 + pm, LANES)] = row_v
            else:
                vv = vocab[d]
                rpart = ((idx >> 7) << 10) + (idx & 127)
                di = STREAM_DEPTHS.index(d)
                for e in range(EMBED_DIM):
                    a_c = (e // 8) * (vv * 8) + (e % 8) * 128
                    pidx[di][pl.ds(e * PTS_PER_WORKER + p0,
                                   LANES)] = rpart + a_c

    copies = []
    for di, d in enumerate(STREAM_DEPTHS):
        for e in range(EMBED_DIM):
            row = (d - 1) * EMBED_DIM + e
            for ct in range(CT_PER_WORKER):
                idx_ref = pidx[di].at[pl.ds(e * PTS_PER_WORKER + ct * 128,
                                            128)]
                dst = out_v.at[pl.ds(_pos(row, ct), 128)]
                copies.append(
                    pltpu.async_copy(flats[di].at[idx_ref], dst, gsem))
    for cp in copies:
        cp.wait()

    for rg in range(ROW_GROUPS):
        for ct in range(CT_PER_WORKER):
            src = out_v.at[pl.ds((rg * CT_PER_WORKER + ct) * 1024, 1024)]
            gct = wid * CT_PER_WORKER + ct
            pltpu.sync_copy(
                src, out_flat.at[pl.ds((rg * N_COL_TILES + gct) * 1024,
                                       1024)])


@jax.jit
def kernel(input, W1, W2, W3, W4, W5, W6, W7):
    tables = [W1, W2, W3, W4, W5, W6, W7]
    wts = [w.T for w in tables]
    flats = [_flat_tile_view(wts[d - 1]) for d in STREAM_DEPTHS]

    mesh = plsc.VectorSubcoreMesh(
        core_axis_name="c", subcore_axis_name="s",
        num_cores=NUM_CORES, num_subcores=NUM_SUBCORES)
    f = pl.kernel(
        _body,
        out_type=jax.ShapeDtypeStruct((OUT_DIM * N_POINTS,), jnp.float32),
        mesh=mesh,
        scratch_types=[
            pltpu.VMEM((3, PTS_PER_WORKER), jnp.float32),
            pltpu.VMEM((8, EMBED_DIM), jnp.float32),
            pltpu.VMEM((EMBED_DIM, 64), jnp.float32),
        ] + [pltpu.VMEM((EMBED_DIM * PTS_PER_WORKER,), jnp.int32)
             for _ in range(N_STREAM)] + [
            pltpu.VMEM((OUT_DIM * PTS_PER_WORKER,), jnp.float32),
            pltpu.SemaphoreType.DMA,
        ],
        compiler_params=pltpu.CompilerParams(needs_layout_passes=False),
    )
    out_flat = f(input.T, W1, wts[1], *flats)
    return (out_flat.reshape(ROW_GROUPS, N_COL_TILES, 8, 128)
            .transpose(0, 2, 1, 3)
            .reshape(OUT_DIM, N_POINTS)
            .T)


# speedup vs baseline: 6.2623x; 1.0800x over previous
import jax
import jax.numpy as jnp
from jax import lax
from jax.experimental import pallas as pl
from jax.experimental.pallas import tpu as pltpu
from jax.experimental.pallas import tpu_sc as plsc

N_DEPTH = 7
EMBED_DIM = 16
N_POINTS = 16384
OUT_DIM = N_DEPTH * EMBED_DIM
LANES = 16

NUM_CORES = 2
NUM_SUBCORES = 16
NUM_WORKERS = NUM_CORES * NUM_SUBCORES
PTS_PER_WORKER = N_POINTS // NUM_WORKERS
GROUPS = PTS_PER_WORKER // LANES
CT_PER_WORKER = PTS_PER_WORKER // 128
ROW_GROUPS = OUT_DIM // 8
N_COL_TILES = N_POINTS // 128

STREAM_DEPTHS = list(range(3, N_DEPTH + 1))
LOCAL_DEPTHS = [1, 2]
N_STREAM = len(STREAM_DEPTHS)

_ONE_MINUS_EPS = 1.0 - 1e-06


def _flat_tile_view(wt):
    v = wt.shape[1]
    return (wt.reshape(2, 8, v // 128, 128)
              .transpose(0, 2, 1, 3)
              .reshape(-1))


def _pos(row, ct):
    return ((row // 8) * CT_PER_WORKER + ct) * 1024 + (row % 8) * 128


def _body(inp_t, w1, wt2, f3, f4, f5, f6, f7, out_flat,
          coords_v, t1_v, t2_v, p3, p4, p5, p6, p7, out_v, gsem):
    flats = [f3, f4, f5, f6, f7]
    pidx = [p3, p4, p5, p6, p7]
    vocab = {d: (2 ** d) ** 3 for d in range(1, N_DEPTH + 1)}

    wid = lax.axis_index("s") * NUM_CORES + lax.axis_index("c")
    base = wid * PTS_PER_WORKER

    pltpu.sync_copy(inp_t.at[:, pl.ds(base, PTS_PER_WORKER)], coords_v)
    pltpu.sync_copy(w1, t1_v)
    pltpu.sync_copy(wt2, t2_v)

    @pl.loop(0, GROUPS)
    def _grp(i):
        p0 = i * LANES
        ct = i // 8
        pm = (i % 8) * LANES
        cell = []
        for a in range(3):
            v = coords_v[a, pl.ds(p0, LANES)]
            s = (v + 1.0) * 0.5
            s = jnp.minimum(jnp.maximum(s, 0.0), _ONE_MINUS_EPS)
            cell.append((s * 128.0).astype(jnp.int32))
        x7, y7, z7 = cell
        for d in range(1, N_DEPTH + 1):
            sh = N_DEPTH - d
            idx = (((z7 >> sh) << (2 * d))
                   + ((y7 >> sh) << d)
                   + (x7 >> sh))
            if d in LOCAL_DEPTHS:
                for e in range(EMBED_DIM):
                    if d == 1:
                        row_v = plsc.load_gather(
                            t1_v, [idx, jnp.full((LANES,), e, jnp.int32)])
                    else:
                        row_v = plsc.load_gather(
                            t2_v, [jnp.full((LANES,), e, jnp.int32), idx])
                    row = (d - 1) * EMBED_DIM + e
                    off = (((row // 8) * CT_PER_WORKER) * 1024
                           + (row % 8) * 128)
                    out_v[pl.ds(off + ct * 1024 + pm, LANES)] = row_v
            else:
                vv = vocab[d]
                rpart = ((idx >> 7) << 10) + (idx & 127)
                di = STREAM_DEPTHS.index(d)
                for e in range(EMBED_DIM):
                    a_c = (e // 8) * (vv * 8) + (e % 8) * 128
                    off = ((e // 8) * 4096 + (e % 8) * 128
                           + ct * 1024 + pm)
                    pidx[di][pl.ds(off, LANES)] = rpart + a_c

    copies = []
    for di, d in enumerate(STREAM_DEPTHS):
        for h in range(2):
            idx_ref = pidx[di].at[pl.ds(h * 4096, 4096)]
            rg = 2 * (d - 1) + h
            dst = out_v.at[pl.ds(rg * CT_PER_WORKER * 1024, 4096)]
            copies.append(
                pltpu.async_copy(flats[di].at[idx_ref], dst, gsem))
    for cp in copies:
        cp.wait()

    for rg in range(ROW_GROUPS):
        for ct in range(CT_PER_WORKER):
            src = out_v.at[pl.ds((rg * CT_PER_WORKER + ct) * 1024, 1024)]
            gct = wid * CT_PER_WORKER + ct
            pltpu.sync_copy(
                src, out_flat.at[pl.ds((rg * N_COL_TILES + gct) * 1024,
                                       1024)])


@jax.jit
def kernel(input, W1, W2, W3, W4, W5, W6, W7):
    tables = [W1, W2, W3, W4, W5, W6, W7]
    wts = [w.T for w in tables]
    flats = [_flat_tile_view(wts[d - 1]) for d in STREAM_DEPTHS]

    mesh = plsc.VectorSubcoreMesh(
        core_axis_name="c", subcore_axis_name="s",
        num_cores=NUM_CORES, num_subcores=NUM_SUBCORES)
    f = pl.kernel(
        _body,
        out_type=jax.ShapeDtypeStruct((OUT_DIM * N_POINTS,), jnp.float32),
        mesh=mesh,
        scratch_types=[
            pltpu.VMEM((3, PTS_PER_WORKER), jnp.float32),
            pltpu.VMEM((8, EMBED_DIM), jnp.float32),
            pltpu.VMEM((EMBED_DIM, 64), jnp.float32),
        ] + [pltpu.VMEM((EMBED_DIM * PTS_PER_WORKER,), jnp.int32)
             for _ in range(N_STREAM)] + [
            pltpu.VMEM((OUT_DIM * PTS_PER_WORKER,), jnp.float32),
            pltpu.SemaphoreType.DMA,
        ],
        compiler_params=pltpu.CompilerParams(needs_layout_passes=False),
    )
    out_flat = f(input.T, W1, wts[1], *flats)
    return (out_flat.reshape(ROW_GROUPS, N_COL_TILES, 8, 128)
            .transpose(0, 2, 1, 3)
            .reshape(OUT_DIM, N_POINTS)
            .T)
